# baseline (speedup 1.0000x reference)
import functools

import jax
import jax.numpy as jnp
from jax import lax
from jax.experimental import pallas as pl
from jax.experimental.pallas import tpu as pltpu
from jax.experimental.pallas import tpu_sc as plsc

N = 10000
E = 320000
D = 128

NC = 2
NS = 16
NW = NC * NS
CHUNK = 64
CH = 160
PW = CH * CHUNK
EP = NW * PW
NBUF = 1
GB = 4
LA = 1
WCH = 40
NWIN = CH // WCH
ACC_ROWS = 10240
DUMMY_COL = 10200
SHIFT = 14
NQ = CHUNK // 16

_mesh = plsc.VectorSubcoreMesh(core_axis_name="c", subcore_axis_name="s")

_ZROWS_PER_TILE = ACC_ROWS // NS


def _unpack_chunk(pk_v, j, row_r, col_r):
    for q in range(NQ):
        v = pk_v[j, pl.ds(q * 16, 16)]
        row_r[pl.ds(q * 16, 16)] = lax.shift_right_logical(v, SHIFT)
        col_r[pl.ds(q * 16, 16)] = lax.bitwise_and(v, (1 << SHIFT) - 1)


@functools.partial(
    pl.kernel,
    out_type=jax.ShapeDtypeStruct((NC, ACC_ROWS, D), jnp.float32),
    mesh=_mesh,
    scratch_types=(
        [pltpu.VMEM((WCH, CHUNK), jnp.int32),
         pltpu.VMEM_SHARED((ACC_ROWS, D), jnp.float32)]
        + [pltpu.VMEM((CHUNK, D), jnp.float32) for _ in range(GB)]
        + [pltpu.VMEM((CHUNK,), jnp.int32) for _ in range(2 * GB)]
        + [pltpu.SemaphoreType.DMA for _ in range(2 * GB)]
    ),
)
def _spmm_sc(g_hbm, pk_hbm, zeros_hbm, out_hbm, pkw, acc_sh, *rest):
    bufs = rest[0:GB]
    rowr = rest[GB:2 * GB]
    colr = rest[2 * GB:3 * GB]
    gsem = rest[3 * GB:4 * GB]
    ssem = rest[4 * GB:]
    c = lax.axis_index("c")
    s = lax.axis_index("s")
    w = c * NS + s

    def fill(b, j):
        _unpack_chunk(pkw, j, rowr[b], colr[b])
        pltpu.async_copy(g_hbm.at[rowr[b]], bufs[b], gsem[b])

    def refill(b, j):
        pltpu.make_async_copy(bufs[b], acc_sh.at[colr[b]], ssem[b]).wait()
        fill(b, j)

    pltpu.sync_copy(pk_hbm.at[w, pl.ds(0, WCH)], pkw)
    for b in range(LA):
        fill(b, b)

    zbase = s * _ZROWS_PER_TILE
    pltpu.sync_copy(zeros_hbm.at[pl.ds(zbase, _ZROWS_PER_TILE)],
                    acc_sh.at[pl.ds(zbase, _ZROWS_PER_TILE)])

    plsc.subcore_barrier()

    def slot(k, first_window):
        b = k % GB
        pltpu.make_async_copy(g_hbm.at[rowr[b]], bufs[b], gsem[b]).wait()
        pltpu.async_copy(bufs[b], acc_sh.at[colr[b]], ssem[b], add=True)
        if k + LA < WCH:
            b2 = (k + LA) % GB
            if first_window and k + LA < GB:
                fill(b2, k + LA)
            else:
                refill(b2, k + LA)

    for k in range(WCH):
        slot(k, True)

    def window(win, carry):
        pltpu.sync_copy(pk_hbm.at[w, pl.ds(win * WCH, WCH)], pkw)
        for j in range(LA):
            refill(j % GB, j)
        for k in range(WCH):
            slot(k, False)
        return carry

    lax.fori_loop(1, NWIN, window, 0)

    for b in range(GB):
        pltpu.make_async_copy(bufs[b], acc_sh.at[colr[b]], ssem[b]).wait()

    plsc.subcore_barrier()

    pltpu.sync_copy(acc_sh.at[pl.ds(zbase, _ZROWS_PER_TILE)],
                    out_hbm.at[c, pl.ds(zbase, _ZROWS_PER_TILE)])


@functools.partial(
    pl.kernel,
    out_type=jax.ShapeDtypeStruct((NC, ACC_ROWS), jnp.float32),
    mesh=_mesh,
    scratch_types=(
        [pltpu.VMEM((CH, CHUNK), jnp.int32),
         pltpu.VMEM((CHUNK,), jnp.float32),
         pltpu.VMEM((CHUNK,), jnp.int32),
         pltpu.VMEM_SHARED((ACC_ROWS,), jnp.float32)]
        + [pltpu.VMEM((CHUNK,), jnp.int32) for _ in range(NBUF)]
        + [pltpu.SemaphoreType.DMA for _ in range(NBUF)]
    ),
)
def _deg_sc(pk_hbm, ones_hbm, zeros1_hbm, out_hbm, pk_v, ones_v, row_x,
            acc_sh, *rest):
    colr = rest[:NBUF]
    ssem = rest[NBUF:]
    c = lax.axis_index("c")
    s = lax.axis_index("s")
    w = c * NS + s

    zbase = s * _ZROWS_PER_TILE
    pltpu.sync_copy(zeros1_hbm.at[pl.ds(zbase, _ZROWS_PER_TILE)],
                    acc_sh.at[pl.ds(zbase, _ZROWS_PER_TILE)])
    pltpu.sync_copy(pk_hbm.at[w], pk_v)
    pltpu.sync_copy(ones_hbm, ones_v)

    plsc.subcore_barrier()

    def outer(o, carry):
        for k in range(NBUF):
            t = o * NBUF + k

            @pl.when(t >= NBUF)
            def _():
                pltpu.make_async_copy(ones_v, acc_sh.at[colr[k]],
                                      ssem[k]).wait()

            _unpack_chunk(pk_v, t, row_x, colr[k])
            pltpu.async_copy(ones_v, acc_sh.at[colr[k]], ssem[k], add=True)
        return carry

    lax.fori_loop(0, CH // NBUF, outer, 0)

    for b in range(NBUF):
        pltpu.make_async_copy(ones_v, acc_sh.at[colr[b]], ssem[b]).wait()

    plsc.subcore_barrier()

    pltpu.sync_copy(acc_sh.at[pl.ds(zbase, _ZROWS_PER_TILE)],
                    out_hbm.at[c, pl.ds(zbase, _ZROWS_PER_TILE)])


R = 1000
GRID = N // R


def _rowspec(d):
    return pl.BlockSpec((R, d), lambda i: (i, 0))


def _pairspec(d):
    return pl.BlockSpec((NC, R, d), lambda i: (0, i, 0))


def _fullspec(shape):
    nd = len(shape)
    return pl.BlockSpec(shape, lambda i, _n=nd: (0,) * _n)


def _colspec():
    return pl.BlockSpec((R, 1), lambda i: (i, 0))


def _matmul_t(a, w):
    return lax.dot_general(a, w, (((1,), (1,)), ((), ())),
                           preferred_element_type=jnp.float32)


def _tc_deg_body(degp, dis_o):
    deg = degp[0] + degp[1] + 1.0
    dis_o[...] = lax.rsqrt(deg)[:, None]


def _tc_deg(degp):
    return pl.pallas_call(
        _tc_deg_body,
        grid=(1,),
        in_specs=[_fullspec((NC, ACC_ROWS))],
        out_specs=_fullspec((ACC_ROWS, 1)),
        out_shape=jax.ShapeDtypeStruct((ACC_ROWS, 1), jnp.float32),
    )(degp)


def _tc_a_body(dis, x, w1, b1, xs_o, t0_o):
    xs_o[...] = x[...] * dis[...]
    t0_o[...] = _matmul_t(x[...], w1[...]) + b1[...][None, :]


def _tc_a(dis, x, w1, b1):
    return pl.pallas_call(
        _tc_a_body,
        grid=(GRID,),
        in_specs=[_colspec(), _rowspec(D),
                  _fullspec((D, D)), _fullspec((D,))],
        out_specs=[_rowspec(D), _rowspec(D)],
        out_shape=[jax.ShapeDtypeStruct((N, D), jnp.float32),
                   jax.ShapeDtypeStruct((N, D), jnp.float32)],
    )(dis, x, w1, b1)


def _tc_b_body(p1, dis, xs, w1, b1, t1_o, ys_o):
    y1 = (p1[0] + p1[1] + xs[...]) * dis[...]
    t1_o[...] = _matmul_t(y1, w1[...]) + b1[...][None, :]
    ys_o[...] = y1 * dis[...]


def _tc_b(p1, dis, xs, w1, b1):
    return pl.pallas_call(
        _tc_b_body,
        grid=(GRID,),
        in_specs=[_pairspec(D), _colspec(), _rowspec(D),
                  _fullspec((D, D)), _fullspec((D,))],
        out_specs=[_rowspec(D), _rowspec(D)],
        out_shape=[jax.ShapeDtypeStruct((N, D), jnp.float32),
                   jax.ShapeDtypeStruct((N, D), jnp.float32)],
    )(p1, dis, xs, w1, b1)


def _tc_c_body(p2, dis, ys, t0, t1, w12, b12, w20, b20, w21, b21, w22, b22,
               u0_o, z1s_o, z2s_o):
    disv = dis[...]
    y2 = (p2[0] + p2[1] + ys[...]) * disv
    t2 = _matmul_t(y2, w12[...]) + b12[...][None, :]
    h = jnp.maximum(jnp.concatenate([t0[...], t1[...], t2], axis=1), 0.0)
    u0_o[...] = _matmul_t(h, w20[...]) + b20[...][None, :]
    z1s_o[...] = (_matmul_t(h, w21[...]) + b21[...][None, :]) * disv
    z2s_o[...] = (_matmul_t(h, w22[...]) + b22[...][None, :]) * disv


def _tc_c(p2, dis, ys, t0, t1, w12, b12, w20, b20, w21, b21, w22, b22):
    return pl.pallas_call(
        _tc_c_body,
        grid=(GRID,),
        in_specs=[_pairspec(D), _colspec(), _rowspec(D), _rowspec(D),
                  _rowspec(D), _fullspec((D, D)), _fullspec((D,)),
                  _fullspec((D, 3 * D)), _fullspec((D,)),
                  _fullspec((D, 3 * D)), _fullspec((D,)),
                  _fullspec((D, 3 * D)), _fullspec((D,))],
        out_specs=[_rowspec(D), _rowspec(D), _rowspec(D)],
        out_shape=[jax.ShapeDtypeStruct((N, D), jnp.float32),
                   jax.ShapeDtypeStruct((N, D), jnp.float32),
                   jax.ShapeDtypeStruct((N, D), jnp.float32)],
    )(p2, dis, ys, t0, t1, w12, b12, w20, b20, w21, b21, w22, b22)


def _tc_d_body(p3, p4, dis, z1s, z2s, u1_o, vs_o):
    disv = dis[...]
    u1_o[...] = (p3[0] + p3[1] + z1s[...]) * disv
    vs_o[...] = (p4[0] + p4[1] + z2s[...]) * disv * disv


def _tc_d(p3, p4, dis, z1s, z2s):
    return pl.pallas_call(
        _tc_d_body,
        grid=(GRID,),
        in_specs=[_pairspec(D), _pairspec(D), _colspec(), _rowspec(D),
                  _rowspec(D)],
        out_specs=[_rowspec(D), _rowspec(D)],
        out_shape=[jax.ShapeDtypeStruct((N, D), jnp.float32),
                   jax.ShapeDtypeStruct((N, D), jnp.float32)],
    )(p3, p4, dis, z1s, z2s)


def _tc_e_body(p5, dis, vs, u0, u1, wf, bf, out_o):
    u2 = (p5[0] + p5[1] + vs[...]) * dis[...]
    wfm = wf[...]
    acc = _matmul_t(u0[...], wfm[:, 0:D])
    acc = acc + _matmul_t(u1[...], wfm[:, D:2 * D])
    acc = acc + _matmul_t(u2, wfm[:, 2 * D:3 * D])
    out_o[...] = acc + bf[...][None, :]


def _tc_e(p5, dis, vs, u0, u1, wf, bf):
    return pl.pallas_call(
        _tc_e_body,
        grid=(GRID,),
        in_specs=[_pairspec(D), _colspec(), _rowspec(D), _rowspec(D),
                  _rowspec(D), _fullspec((D, 3 * D)), _fullspec((D,))],
        out_specs=_rowspec(D),
        out_shape=jax.ShapeDtypeStruct((N, D), jnp.float32),
    )(p5, dis, vs, u0, u1, wf, bf)


def kernel(x, edge_index, W1_0, b1_0, W1_1, b1_1, W1_2, b1_2,
           W2_0, b2_0, W2_1, b2_1, W2_2, b2_2, Wf, bf):
    row = edge_index[0].astype(jnp.int32)
    col = edge_index[1].astype(jnp.int32)
    npad = EP - E
    packed = (row << SHIFT) | col
    pad_i = jnp.arange(npad, dtype=jnp.int32)
    pad_pk = (((pad_i * 197) % N) << SHIFT) | (N + pad_i % (ACC_ROWS - N))
    packed = jnp.concatenate([packed, pad_pk])
    packed = packed.reshape(CH * CHUNK, NW).T.reshape(NW, CH, CHUNK)

    zeros2 = jnp.zeros((ACC_ROWS, D), jnp.float32)
    zeros1 = jnp.zeros((ACC_ROWS,), jnp.float32)
    ones1 = jnp.ones((CHUNK,), jnp.float32)

    degp = _deg_sc(packed, ones1, zeros1)
    dis = _tc_deg(degp)
    xs, t0 = _tc_a(dis, x, W1_0, b1_0)
    p1 = _spmm_sc(xs, packed, zeros2)
    t1, ys = _tc_b(p1, dis, xs, W1_1, b1_1)
    p2 = _spmm_sc(ys, packed, zeros2)
    u0, z1s, z2s = _tc_c(p2, dis, ys, t0, t1, W1_2, b1_2,
                         W2_0, b2_0, W2_1, b2_1, W2_2, b2_2)
    p3 = _spmm_sc(z1s, packed, zeros2)
    p4 = _spmm_sc(z2s, packed, zeros2)
    u1, vs = _tc_d(p3, p4, dis, z1s, z2s)
    p5 = _spmm_sc(vs, packed, zeros2)
    out = _tc_e(p5, dis, vs, u0, u1, Wf, bf)
    return out


# speedup vs baseline: 1.3988x; 1.3988x over previous
import functools

import jax
import jax.numpy as jnp
from jax import lax
from jax.experimental import pallas as pl
from jax.experimental.pallas import tpu as pltpu
from jax.experimental.pallas import tpu_sc as plsc

N = 10000
E = 320000
D = 128

NC = 2
NS = 16
NW = NC * NS
CHUNK = 64
CH = 160
PW = CH * CHUNK
EP = NW * PW
NBUF = 1
GB = 4
LA = 2
WCH = 40
NWIN = CH // WCH
ACC_ROWS = 10240
DUMMY_COL = 10200
SHIFT = 14
NQ = CHUNK // 16

_mesh = plsc.VectorSubcoreMesh(core_axis_name="c", subcore_axis_name="s")

_ZROWS_PER_TILE = ACC_ROWS // NS


def _unpack_chunk(pk_v, j, row_r, col_r, nq=NQ):
    for q in range(nq):
        v = pk_v[j, pl.ds(q * 16, 16)]
        row_r[pl.ds(q * 16, 16)] = lax.shift_right_logical(v, SHIFT)
        col_r[pl.ds(q * 16, 16)] = lax.bitwise_and(v, (1 << SHIFT) - 1)


def _spmm_pipeline(g_hbm, out_hbm, pk_hbm, zeros_hbm, pkw, acc_sh,
                   bufs, rowr, colr, gsem, ssem, w, c, s):
    zbase = s * _ZROWS_PER_TILE

    def fill(b, j):
        _unpack_chunk(pkw, j, rowr[b], colr[b])
        pltpu.async_copy(g_hbm.at[rowr[b]], bufs[b], gsem[b])

    def refill(b, j):
        pltpu.make_async_copy(bufs[b], acc_sh.at[colr[b]], ssem[b]).wait()
        fill(b, j)

    pltpu.sync_copy(pk_hbm.at[w, pl.ds(0, WCH)], pkw)
    for b in range(LA):
        fill(b, b)

    pltpu.sync_copy(zeros_hbm.at[pl.ds(zbase, _ZROWS_PER_TILE)],
                    acc_sh.at[pl.ds(zbase, _ZROWS_PER_TILE)])

    plsc.subcore_barrier()

    def slot(k, first_window):
        b = k % GB
        pltpu.make_async_copy(g_hbm.at[rowr[b]], bufs[b], gsem[b]).wait()
        pltpu.async_copy(bufs[b], acc_sh.at[colr[b]], ssem[b], add=True)
        if k + LA < WCH:
            b2 = (k + LA) % GB
            if first_window and k + LA < GB:
                fill(b2, k + LA)
            else:
                refill(b2, k + LA)

    for k in range(WCH):
        slot(k, True)

    def window(win, carry):
        pltpu.sync_copy(pk_hbm.at[w, pl.ds(win * WCH, WCH)], pkw)
        for j in range(LA):
            refill(j % GB, j)
        for k in range(WCH):
            slot(k, False)
        return carry

    lax.fori_loop(1, NWIN, window, 0)

    for b in range(GB):
        pltpu.make_async_copy(bufs[b], acc_sh.at[colr[b]], ssem[b]).wait()

    plsc.subcore_barrier()

    pltpu.sync_copy(acc_sh.at[pl.ds(zbase, _ZROWS_PER_TILE)],
                    out_hbm.at[c, pl.ds(zbase, _ZROWS_PER_TILE)])


_SPMM_SCRATCH = (
    [pltpu.VMEM((WCH, CHUNK), jnp.int32),
     pltpu.VMEM_SHARED((ACC_ROWS, D), jnp.float32)]
    + [pltpu.VMEM((CHUNK, D), jnp.float32) for _ in range(GB)]
    + [pltpu.VMEM((CHUNK,), jnp.int32) for _ in range(2 * GB)]
    + [pltpu.SemaphoreType.DMA for _ in range(2 * GB)]
)


@functools.partial(
    pl.kernel,
    out_type=jax.ShapeDtypeStruct((NC, ACC_ROWS, D), jnp.float32),
    mesh=_mesh,
    scratch_types=_SPMM_SCRATCH,
)
def _spmm_sc(g_hbm, pk_hbm, zeros_hbm, out_hbm, pkw, acc_sh, *rest):
    bufs = rest[0:GB]
    rowr = rest[GB:2 * GB]
    colr = rest[2 * GB:3 * GB]
    gsem = rest[3 * GB:4 * GB]
    ssem = rest[4 * GB:]
    c = lax.axis_index("c")
    s = lax.axis_index("s")
    w = c * NS + s
    _spmm_pipeline(g_hbm, out_hbm, pk_hbm, zeros_hbm, pkw, acc_sh,
                   bufs, rowr, colr, gsem, ssem, w, c, s)


@functools.partial(
    pl.kernel,
    out_type=[jax.ShapeDtypeStruct((NC, ACC_ROWS, D), jnp.float32),
              jax.ShapeDtypeStruct((NC, ACC_ROWS, D), jnp.float32)],
    mesh=_mesh,
    scratch_types=_SPMM_SCRATCH,
)
def _spmm2_sc(g1_hbm, g2_hbm, pk_hbm, zeros_hbm, out1_hbm, out2_hbm,
              pkw, acc_sh, *rest):
    bufs = rest[0:GB]
    rowr = rest[GB:2 * GB]
    colr = rest[2 * GB:3 * GB]
    gsem = rest[3 * GB:4 * GB]
    ssem = rest[4 * GB:]
    c = lax.axis_index("c")
    s = lax.axis_index("s")
    w = c * NS + s
    _spmm_pipeline(g1_hbm, out1_hbm, pk_hbm, zeros_hbm, pkw, acc_sh,
                   bufs, rowr, colr, gsem, ssem, w, c, s)
    _spmm_pipeline(g2_hbm, out2_hbm, pk_hbm, zeros_hbm, pkw, acc_sh,
                   bufs, rowr, colr, gsem, ssem, w, c, s)


DCHUNK = 128
DCH = PW // DCHUNK
DNB = 2


@functools.partial(
    pl.kernel,
    out_type=jax.ShapeDtypeStruct((NC, ACC_ROWS), jnp.float32),
    mesh=_mesh,
    scratch_types=(
        [pltpu.VMEM((DCH, DCHUNK), jnp.int32),
         pltpu.VMEM((DCHUNK,), jnp.float32),
         pltpu.VMEM((DCHUNK,), jnp.int32),
         pltpu.VMEM_SHARED((ACC_ROWS,), jnp.float32)]
        + [pltpu.VMEM((DCHUNK,), jnp.int32) for _ in range(DNB)]
        + [pltpu.SemaphoreType.DMA for _ in range(DNB)]
    ),
)
def _deg_sc(pk_hbm, ones_hbm, zeros1_hbm, out_hbm, pk_v, ones_v, row_x,
            acc_sh, *rest):
    colr = rest[:DNB]
    ssem = rest[DNB:]
    c = lax.axis_index("c")
    s = lax.axis_index("s")
    w = c * NS + s

    zbase = s * _ZROWS_PER_TILE
    pltpu.sync_copy(zeros1_hbm.at[pl.ds(zbase, _ZROWS_PER_TILE)],
                    acc_sh.at[pl.ds(zbase, _ZROWS_PER_TILE)])
    pltpu.sync_copy(pk_hbm.at[w], pk_v)
    pltpu.sync_copy(ones_hbm, ones_v)

    plsc.subcore_barrier()

    def outer(o, carry):
        for k in range(DNB):
            t = o * DNB + k

            @pl.when(t >= DNB)
            def _():
                pltpu.make_async_copy(ones_v, acc_sh.at[colr[k]],
                                      ssem[k]).wait()

            _unpack_chunk(pk_v, t, row_x, colr[k], nq=DCHUNK // 16)
            pltpu.async_copy(ones_v, acc_sh.at[colr[k]], ssem[k], add=True)
        return carry

    lax.fori_loop(0, DCH // DNB, outer, 0)

    for b in range(DNB):
        pltpu.make_async_copy(ones_v, acc_sh.at[colr[b]], ssem[b]).wait()

    plsc.subcore_barrier()

    pltpu.sync_copy(acc_sh.at[pl.ds(zbase, _ZROWS_PER_TILE)],
                    out_hbm.at[c, pl.ds(zbase, _ZROWS_PER_TILE)])


R = 1000
GRID = N // R


def _rowspec(d):
    return pl.BlockSpec((R, d), lambda i: (i, 0))


def _pairspec(d):
    return pl.BlockSpec((NC, R, d), lambda i: (0, i, 0))


def _fullspec(shape):
    nd = len(shape)
    return pl.BlockSpec(shape, lambda i, _n=nd: (0,) * _n)


def _colspec():
    return pl.BlockSpec((R, 1), lambda i: (i, 0))


def _matmul_t(a, w):
    return lax.dot_general(a, w, (((1,), (1,)), ((), ())),
                           preferred_element_type=jnp.float32)


def _tc_deg_body(degp, dis_o):
    deg = degp[0] + degp[1] + 1.0
    dis_o[...] = lax.rsqrt(deg)[:, None]


def _tc_deg(degp):
    return pl.pallas_call(
        _tc_deg_body,
        grid=(1,),
        in_specs=[_fullspec((NC, ACC_ROWS))],
        out_specs=_fullspec((ACC_ROWS, 1)),
        out_shape=jax.ShapeDtypeStruct((ACC_ROWS, 1), jnp.float32),
    )(degp)


def _tc_a_body(dis, x, w1, b1, xs_o, t0_o):
    xs_o[...] = x[...] * dis[...]
    t0_o[...] = _matmul_t(x[...], w1[...]) + b1[...][None, :]


def _tc_a(dis, x, w1, b1):
    return pl.pallas_call(
        _tc_a_body,
        grid=(GRID,),
        in_specs=[_colspec(), _rowspec(D),
                  _fullspec((D, D)), _fullspec((D,))],
        out_specs=[_rowspec(D), _rowspec(D)],
        out_shape=[jax.ShapeDtypeStruct((N, D), jnp.float32),
                   jax.ShapeDtypeStruct((N, D), jnp.float32)],
    )(dis, x, w1, b1)


def _tc_b_body(p1, dis, xs, w1, b1, t1_o, ys_o):
    y1 = (p1[0] + p1[1] + xs[...]) * dis[...]
    t1_o[...] = _matmul_t(y1, w1[...]) + b1[...][None, :]
    ys_o[...] = y1 * dis[...]


def _tc_b(p1, dis, xs, w1, b1):
    return pl.pallas_call(
        _tc_b_body,
        grid=(GRID,),
        in_specs=[_pairspec(D), _colspec(), _rowspec(D),
                  _fullspec((D, D)), _fullspec((D,))],
        out_specs=[_rowspec(D), _rowspec(D)],
        out_shape=[jax.ShapeDtypeStruct((N, D), jnp.float32),
                   jax.ShapeDtypeStruct((N, D), jnp.float32)],
    )(p1, dis, xs, w1, b1)


def _tc_c_body(p2, dis, ys, t0, t1, w12, b12, w20, b20, w21, b21, w22, b22,
               u0_o, z1s_o, z2s_o):
    disv = dis[...]
    y2 = (p2[0] + p2[1] + ys[...]) * disv
    t2 = _matmul_t(y2, w12[...]) + b12[...][None, :]
    h = jnp.maximum(jnp.concatenate([t0[...], t1[...], t2], axis=1), 0.0)
    u0_o[...] = _matmul_t(h, w20[...]) + b20[...][None, :]
    z1s_o[...] = (_matmul_t(h, w21[...]) + b21[...][None, :]) * disv
    z2s_o[...] = (_matmul_t(h, w22[...]) + b22[...][None, :]) * disv


def _tc_c(p2, dis, ys, t0, t1, w12, b12, w20, b20, w21, b21, w22, b22):
    return pl.pallas_call(
        _tc_c_body,
        grid=(GRID,),
        in_specs=[_pairspec(D), _colspec(), _rowspec(D), _rowspec(D),
                  _rowspec(D), _fullspec((D, D)), _fullspec((D,)),
                  _fullspec((D, 3 * D)), _fullspec((D,)),
                  _fullspec((D, 3 * D)), _fullspec((D,)),
                  _fullspec((D, 3 * D)), _fullspec((D,))],
        out_specs=[_rowspec(D), _rowspec(D), _rowspec(D)],
        out_shape=[jax.ShapeDtypeStruct((N, D), jnp.float32),
                   jax.ShapeDtypeStruct((N, D), jnp.float32),
                   jax.ShapeDtypeStruct((N, D), jnp.float32)],
    )(p2, dis, ys, t0, t1, w12, b12, w20, b20, w21, b21, w22, b22)


def _tc_d_body(p3, p4, dis, z1s, z2s, u1_o, vs_o):
    disv = dis[...]
    u1_o[...] = (p3[0] + p3[1] + z1s[...]) * disv
    vs_o[...] = (p4[0] + p4[1] + z2s[...]) * disv * disv


def _tc_d(p3, p4, dis, z1s, z2s):
    return pl.pallas_call(
        _tc_d_body,
        grid=(GRID,),
        in_specs=[_pairspec(D), _pairspec(D), _colspec(), _rowspec(D),
                  _rowspec(D)],
        out_specs=[_rowspec(D), _rowspec(D)],
        out_shape=[jax.ShapeDtypeStruct((N, D), jnp.float32),
                   jax.ShapeDtypeStruct((N, D), jnp.float32)],
    )(p3, p4, dis, z1s, z2s)


def _tc_e_body(p5, dis, vs, u0, u1, wf, bf, out_o):
    u2 = (p5[0] + p5[1] + vs[...]) * dis[...]
    wfm = wf[...]
    acc = _matmul_t(u0[...], wfm[:, 0:D])
    acc = acc + _matmul_t(u1[...], wfm[:, D:2 * D])
    acc = acc + _matmul_t(u2, wfm[:, 2 * D:3 * D])
    out_o[...] = acc + bf[...][None, :]


def _tc_e(p5, dis, vs, u0, u1, wf, bf):
    return pl.pallas_call(
        _tc_e_body,
        grid=(GRID,),
        in_specs=[_pairspec(D), _colspec(), _rowspec(D), _rowspec(D),
                  _rowspec(D), _fullspec((D, 3 * D)), _fullspec((D,))],
        out_specs=_rowspec(D),
        out_shape=jax.ShapeDtypeStruct((N, D), jnp.float32),
    )(p5, dis, vs, u0, u1, wf, bf)


def kernel(x, edge_index, W1_0, b1_0, W1_1, b1_1, W1_2, b1_2,
           W2_0, b2_0, W2_1, b2_1, W2_2, b2_2, Wf, bf):
    row = edge_index[0].astype(jnp.int32)
    col = edge_index[1].astype(jnp.int32)
    npad = EP - E
    packed = (row << SHIFT) | col
    pad_i = jnp.arange(npad, dtype=jnp.int32)
    pad_pk = (((pad_i * 197) % N) << SHIFT) | (N + pad_i % (ACC_ROWS - N))
    packed = jnp.concatenate([packed, pad_pk])
    packed = packed.reshape(CH * CHUNK, NW).T.reshape(NW, CH, CHUNK)

    zeros2 = jnp.zeros((ACC_ROWS, D), jnp.float32)
    zeros1 = jnp.zeros((ACC_ROWS,), jnp.float32)
    ones1 = jnp.ones((DCHUNK,), jnp.float32)

    degp = _deg_sc(packed.reshape(NW, DCH, DCHUNK), ones1, zeros1)
    dis = _tc_deg(degp)
    xs, t0 = _tc_a(dis, x, W1_0, b1_0)
    p1 = _spmm_sc(xs, packed, zeros2)
    t1, ys = _tc_b(p1, dis, xs, W1_1, b1_1)
    p2 = _spmm_sc(ys, packed, zeros2)
    u0, z1s, z2s = _tc_c(p2, dis, ys, t0, t1, W1_2, b1_2,
                         W2_0, b2_0, W2_1, b2_1, W2_2, b2_2)
    p3, p4 = _spmm2_sc(z1s, z2s, packed, zeros2)
    u1, vs = _tc_d(p3, p4, dis, z1s, z2s)
    p5 = _spmm_sc(vs, packed, zeros2)
    out = _tc_e(p5, dis, vs, u0, u1, Wf, bf)
    return out


# speedup vs baseline: 1.4005x; 1.0012x over previous
import functools

import jax
import jax.numpy as jnp
from jax import lax
from jax.experimental import pallas as pl
from jax.experimental.pallas import tpu as pltpu
from jax.experimental.pallas import tpu_sc as plsc

N = 10000
E = 320000
D = 128

NC = 2
NS = 16
NW = NC * NS
CHUNK = 64
CH = 160
PW = CH * CHUNK
EP = NW * PW
GB = 4
LA = 2
WCH = 40
NWIN = CH // WCH
ACC_ROWS = 10240
SHIFT = 14
NQ = CHUNK // 16

_mesh = plsc.VectorSubcoreMesh(core_axis_name="c", subcore_axis_name="s")

_ZROWS_PER_TILE = ACC_ROWS // NS


def _unpack_chunk(pk_v, j, row_r, col_r, nq=NQ):
    for q in range(nq):
        v = pk_v[j, pl.ds(q * 16, 16)]
        row_r[pl.ds(q * 16, 16)] = lax.shift_right_logical(v, SHIFT)
        col_r[pl.ds(q * 16, 16)] = lax.bitwise_and(v, (1 << SHIFT) - 1)


def _spmm_pipeline(g_hbm, out_hbm, pk_hbm, zeros_hbm, pkw, acc_sh,
                   bufs, rowr, colr, gsem, ssem, w, c, s):
    zbase = s * _ZROWS_PER_TILE

    def fill(b, j):
        _unpack_chunk(pkw, j, rowr[b], colr[b])
        pltpu.async_copy(g_hbm.at[rowr[b]], bufs[b], gsem[b])

    def refill(b, j):
        pltpu.make_async_copy(bufs[b], acc_sh.at[colr[b]], ssem[b]).wait()
        fill(b, j)

    pltpu.sync_copy(pk_hbm.at[w, pl.ds(0, WCH)], pkw)
    for b in range(LA):
        fill(b, b)

    pltpu.sync_copy(zeros_hbm.at[pl.ds(zbase, _ZROWS_PER_TILE)],
                    acc_sh.at[pl.ds(zbase, _ZROWS_PER_TILE)])

    plsc.subcore_barrier()

    def slot(k, first_window):
        b = k % GB
        pltpu.make_async_copy(g_hbm.at[rowr[b]], bufs[b], gsem[b]).wait()
        pltpu.async_copy(bufs[b], acc_sh.at[colr[b]], ssem[b], add=True)
        if k + LA < WCH:
            b2 = (k + LA) % GB
            if first_window and k + LA < GB:
                fill(b2, k + LA)
            else:
                refill(b2, k + LA)

    for k in range(WCH):
        slot(k, True)

    def window(win, carry):
        pltpu.sync_copy(pk_hbm.at[w, pl.ds(win * WCH, WCH)], pkw)
        for j in range(LA):
            refill(j % GB, j)
        for k in range(WCH):
            slot(k, False)
        return carry

    lax.fori_loop(1, NWIN, window, 0)

    for b in range(GB):
        pltpu.make_async_copy(bufs[b], acc_sh.at[colr[b]], ssem[b]).wait()

    plsc.subcore_barrier()

    pltpu.sync_copy(acc_sh.at[pl.ds(zbase, _ZROWS_PER_TILE)],
                    out_hbm.at[c, pl.ds(zbase, _ZROWS_PER_TILE)])


_SPMM_SCRATCH = (
    [pltpu.VMEM((WCH, CHUNK), jnp.int32),
     pltpu.VMEM_SHARED((ACC_ROWS, D), jnp.float32)]
    + [pltpu.VMEM((CHUNK, D), jnp.float32) for _ in range(GB)]
    + [pltpu.VMEM((CHUNK,), jnp.int32) for _ in range(2 * GB)]
    + [pltpu.SemaphoreType.DMA for _ in range(2 * GB)]
)


@functools.partial(
    pl.kernel,
    out_type=jax.ShapeDtypeStruct((NC, ACC_ROWS, D), jnp.float32),
    mesh=_mesh,
    scratch_types=_SPMM_SCRATCH,
)
def _spmm_sc(g_hbm, pk_hbm, zeros_hbm, out_hbm, pkw, acc_sh, *rest):
    bufs = rest[0:GB]
    rowr = rest[GB:2 * GB]
    colr = rest[2 * GB:3 * GB]
    gsem = rest[3 * GB:4 * GB]
    ssem = rest[4 * GB:]
    c = lax.axis_index("c")
    s = lax.axis_index("s")
    w = c * NS + s
    _spmm_pipeline(g_hbm, out_hbm, pk_hbm, zeros_hbm, pkw, acc_sh,
                   bufs, rowr, colr, gsem, ssem, w, c, s)


@functools.partial(
    pl.kernel,
    out_type=[jax.ShapeDtypeStruct((NC, ACC_ROWS, D), jnp.float32),
              jax.ShapeDtypeStruct((NC, ACC_ROWS, D), jnp.float32)],
    mesh=_mesh,
    scratch_types=_SPMM_SCRATCH,
)
def _spmm2_sc(g1_hbm, g2_hbm, pk_hbm, zeros_hbm, out1_hbm, out2_hbm,
              pkw, acc_sh, *rest):
    bufs = rest[0:GB]
    rowr = rest[GB:2 * GB]
    colr = rest[2 * GB:3 * GB]
    gsem = rest[3 * GB:4 * GB]
    ssem = rest[4 * GB:]
    c = lax.axis_index("c")
    s = lax.axis_index("s")
    w = c * NS + s
    _spmm_pipeline(g1_hbm, out1_hbm, pk_hbm, zeros_hbm, pkw, acc_sh,
                   bufs, rowr, colr, gsem, ssem, w, c, s)
    _spmm_pipeline(g2_hbm, out2_hbm, pk_hbm, zeros_hbm, pkw, acc_sh,
                   bufs, rowr, colr, gsem, ssem, w, c, s)


DCHUNK = 128
DCH = PW // DCHUNK
DNB = 2


@functools.partial(
    pl.kernel,
    out_type=jax.ShapeDtypeStruct((NC, ACC_ROWS), jnp.float32),
    mesh=_mesh,
    scratch_types=(
        [pltpu.VMEM((DCH, DCHUNK), jnp.int32),
         pltpu.VMEM((DCHUNK,), jnp.float32),
         pltpu.VMEM((DCHUNK,), jnp.int32),
         pltpu.VMEM_SHARED((ACC_ROWS,), jnp.float32)]
        + [pltpu.VMEM((DCHUNK,), jnp.int32) for _ in range(DNB)]
        + [pltpu.SemaphoreType.DMA for _ in range(DNB)]
    ),
)
def _deg_sc(pk_hbm, ones_hbm, zeros1_hbm, out_hbm, pk_v, ones_v, row_x,
            acc_sh, *rest):
    colr = rest[:DNB]
    ssem = rest[DNB:]
    c = lax.axis_index("c")
    s = lax.axis_index("s")
    w = c * NS + s

    zbase = s * _ZROWS_PER_TILE
    pltpu.sync_copy(zeros1_hbm.at[pl.ds(zbase, _ZROWS_PER_TILE)],
                    acc_sh.at[pl.ds(zbase, _ZROWS_PER_TILE)])
    pltpu.sync_copy(pk_hbm.at[w], pk_v)
    pltpu.sync_copy(ones_hbm, ones_v)

    plsc.subcore_barrier()

    def outer(o, carry):
        for k in range(DNB):
            t = o * DNB + k

            @pl.when(t >= DNB)
            def _():
                pltpu.make_async_copy(ones_v, acc_sh.at[colr[k]],
                                      ssem[k]).wait()

            _unpack_chunk(pk_v, t, row_x, colr[k], nq=DCHUNK // 16)
            pltpu.async_copy(ones_v, acc_sh.at[colr[k]], ssem[k], add=True)
        return carry

    lax.fori_loop(0, DCH // DNB, outer, 0)

    for b in range(DNB):
        pltpu.make_async_copy(ones_v, acc_sh.at[colr[b]], ssem[b]).wait()

    plsc.subcore_barrier()

    pltpu.sync_copy(acc_sh.at[pl.ds(zbase, _ZROWS_PER_TILE)],
                    out_hbm.at[c, pl.ds(zbase, _ZROWS_PER_TILE)])


R = 1000
GRID = N // R


def _rowspec(d):
    return pl.BlockSpec((R, d), lambda i: (i, 0))


def _pairspec(d):
    return pl.BlockSpec((NC, R, d), lambda i: (0, i, 0))


def _fullspec(shape):
    nd = len(shape)
    return pl.BlockSpec(shape, lambda i, _n=nd: (0,) * _n)


def _colspec():
    return pl.BlockSpec((R, 1), lambda i: (i, 0))


def _matmul_t(a, w):
    return lax.dot_general(a, w, (((1,), (1,)), ((), ())),
                           preferred_element_type=jnp.float32)


def _tc_deg_body(degp, dis_o):
    deg = degp[0] + degp[1] + 1.0
    dis_o[...] = lax.rsqrt(deg)[:, None]


def _tc_deg(degp):
    return pl.pallas_call(
        _tc_deg_body,
        grid=(1,),
        in_specs=[_fullspec((NC, ACC_ROWS))],
        out_specs=_fullspec((ACC_ROWS, 1)),
        out_shape=jax.ShapeDtypeStruct((ACC_ROWS, 1), jnp.float32),
    )(degp)


def _tc_a_body(dis, x, w1, b1, xs_o, t0_o):
    xs_o[...] = x[...] * dis[...]
    t0_o[...] = _matmul_t(x[...], w1[...]) + b1[...][None, :]


def _tc_a(dis, x, w1, b1):
    return pl.pallas_call(
        _tc_a_body,
        grid=(GRID,),
        in_specs=[_colspec(), _rowspec(D),
                  _fullspec((D, D)), _fullspec((D,))],
        out_specs=[_rowspec(D), _rowspec(D)],
        out_shape=[jax.ShapeDtypeStruct((N, D), jnp.float32),
                   jax.ShapeDtypeStruct((N, D), jnp.float32)],
    )(dis, x, w1, b1)


def _tc_b_body(p1, dis, xs, w1, b1, t1_o, ys_o):
    y1 = (p1[0] + p1[1] + xs[...]) * dis[...]
    t1_o[...] = _matmul_t(y1, w1[...]) + b1[...][None, :]
    ys_o[...] = y1 * dis[...]


def _tc_b(p1, dis, xs, w1, b1):
    return pl.pallas_call(
        _tc_b_body,
        grid=(GRID,),
        in_specs=[_pairspec(D), _colspec(), _rowspec(D),
                  _fullspec((D, D)), _fullspec((D,))],
        out_specs=[_rowspec(D), _rowspec(D)],
        out_shape=[jax.ShapeDtypeStruct((N, D), jnp.float32),
                   jax.ShapeDtypeStruct((N, D), jnp.float32)],
    )(p1, dis, xs, w1, b1)


def _tc_c_body(p2, dis, ys, t0, t1, w12, b12, w20, b20, w21, b21, w22, b22,
               u0_o, z1s_o, z2s_o):
    disv = dis[...]
    y2 = (p2[0] + p2[1] + ys[...]) * disv
    t2 = _matmul_t(y2, w12[...]) + b12[...][None, :]
    h = jnp.maximum(jnp.concatenate([t0[...], t1[...], t2], axis=1), 0.0)
    u0_o[...] = _matmul_t(h, w20[...]) + b20[...][None, :]
    z1s_o[...] = (_matmul_t(h, w21[...]) + b21[...][None, :]) * disv
    z2s_o[...] = (_matmul_t(h, w22[...]) + b22[...][None, :]) * disv


def _tc_c(p2, dis, ys, t0, t1, w12, b12, w20, b20, w21, b21, w22, b22):
    return pl.pallas_call(
        _tc_c_body,
        grid=(GRID,),
        in_specs=[_pairspec(D), _colspec(), _rowspec(D), _rowspec(D),
                  _rowspec(D), _fullspec((D, D)), _fullspec((D,)),
                  _fullspec((D, 3 * D)), _fullspec((D,)),
                  _fullspec((D, 3 * D)), _fullspec((D,)),
                  _fullspec((D, 3 * D)), _fullspec((D,))],
        out_specs=[_rowspec(D), _rowspec(D), _rowspec(D)],
        out_shape=[jax.ShapeDtypeStruct((N, D), jnp.float32),
                   jax.ShapeDtypeStruct((N, D), jnp.float32),
                   jax.ShapeDtypeStruct((N, D), jnp.float32)],
    )(p2, dis, ys, t0, t1, w12, b12, w20, b20, w21, b21, w22, b22)


def _tc_d_body(p3, p4, dis, z1s, z2s, u1_o, vs_o):
    disv = dis[...]
    u1_o[...] = (p3[0] + p3[1] + z1s[...]) * disv
    vs_o[...] = (p4[0] + p4[1] + z2s[...]) * disv * disv


def _tc_d(p3, p4, dis, z1s, z2s):
    return pl.pallas_call(
        _tc_d_body,
        grid=(GRID,),
        in_specs=[_pairspec(D), _pairspec(D), _colspec(), _rowspec(D),
                  _rowspec(D)],
        out_specs=[_rowspec(D), _rowspec(D)],
        out_shape=[jax.ShapeDtypeStruct((N, D), jnp.float32),
                   jax.ShapeDtypeStruct((N, D), jnp.float32)],
    )(p3, p4, dis, z1s, z2s)


def _tc_e_body(p5, dis, vs, u0, u1, wf, bf, out_o):
    u2 = (p5[0] + p5[1] + vs[...]) * dis[...]
    wfm = wf[...]
    acc = _matmul_t(u0[...], wfm[:, 0:D])
    acc = acc + _matmul_t(u1[...], wfm[:, D:2 * D])
    acc = acc + _matmul_t(u2, wfm[:, 2 * D:3 * D])
    out_o[...] = acc + bf[...][None, :]


def _tc_e(p5, dis, vs, u0, u1, wf, bf):
    return pl.pallas_call(
        _tc_e_body,
        grid=(GRID,),
        in_specs=[_pairspec(D), _colspec(), _rowspec(D), _rowspec(D),
                  _rowspec(D), _fullspec((D, 3 * D)), _fullspec((D,))],
        out_specs=_rowspec(D),
        out_shape=jax.ShapeDtypeStruct((N, D), jnp.float32),
    )(p5, dis, vs, u0, u1, wf, bf)


def kernel(x, edge_index, W1_0, b1_0, W1_1, b1_1, W1_2, b1_2,
           W2_0, b2_0, W2_1, b2_1, W2_2, b2_2, Wf, bf):
    row = edge_index[0].astype(jnp.int32)
    col = edge_index[1].astype(jnp.int32)
    npad = EP - E
    packed = (row << SHIFT) | col
    pad_i = jnp.arange(npad, dtype=jnp.int32)
    pad_pk = (((pad_i * 197) % N) << SHIFT) | (N + pad_i % (ACC_ROWS - N))
    packed = jnp.concatenate([packed, pad_pk])
    packed = packed.reshape(CH * CHUNK, NW).T.reshape(NW, CH, CHUNK)

    zeros2 = jnp.zeros((ACC_ROWS, D), jnp.float32)
    zeros1 = jnp.zeros((ACC_ROWS,), jnp.float32)
    ones1 = jnp.ones((DCHUNK,), jnp.float32)

    degp = _deg_sc(packed.reshape(NW, DCH, DCHUNK), ones1, zeros1)
    dis = _tc_deg(degp)
    xs, t0 = _tc_a(dis, x, W1_0, b1_0)
    p1 = _spmm_sc(xs, packed, zeros2)
    t1, ys = _tc_b(p1, dis, xs, W1_1, b1_1)
    p2 = _spmm_sc(ys, packed, zeros2)
    u0, z1s, z2s = _tc_c(p2, dis, ys, t0, t1, W1_2, b1_2,
                         W2_0, b2_0, W2_1, b2_1, W2_2, b2_2)
    p3, p4 = _spmm2_sc(z1s, z2s, packed, zeros2)
    u1, vs = _tc_d(p3, p4, dis, z1s, z2s)
    p5 = _spmm_sc(vs, packed, zeros2)
    out = _tc_e(p5, dis, vs, u0, u1, Wf, bf)
    return out


# speedup vs baseline: 1.4199x; 1.0139x over previous
import functools

import jax
import jax.numpy as jnp
from jax import lax
from jax.experimental import pallas as pl
from jax.experimental.pallas import tpu as pltpu
from jax.experimental.pallas import tpu_sc as plsc

N = 10000
E = 320000
D = 128

NC = 2
NS = 16
NW = NC * NS
CHUNK = 64
CH = 160
PW = CH * CHUNK
EP = NW * PW
GB = 4
LA = 2
WCH = 40
NWIN = CH // WCH
ACC_ROWS = 10240
SHIFT = 14
NQ = CHUNK // 16

_mesh = plsc.VectorSubcoreMesh(core_axis_name="c", subcore_axis_name="s")

_ZROWS_PER_TILE = ACC_ROWS // NS


def _unpack_chunk(pk_v, j, row_r, col_r, nq=NQ):
    for q in range(nq):
        v = pk_v[j, pl.ds(q * 16, 16)]
        row_r[pl.ds(q * 16, 16)] = lax.shift_right_logical(v, SHIFT)
        col_r[pl.ds(q * 16, 16)] = lax.bitwise_and(v, (1 << SHIFT) - 1)


def _spmm_pipeline(g_hbm, out_hbm, pk_hbm, zeros_hbm, pkw, acc_sh,
                   bufs, rowr, colr, gsem, ssem, w, c, s):
    zbase = s * _ZROWS_PER_TILE

    def fill(b, j):
        _unpack_chunk(pkw, j, rowr[b], colr[b])
        pltpu.async_copy(g_hbm.at[rowr[b]], bufs[b], gsem[b])

    def refill(b, j):
        pltpu.make_async_copy(bufs[b], acc_sh.at[colr[b]], ssem[b]).wait()
        fill(b, j)

    pltpu.sync_copy(pk_hbm.at[w, pl.ds(0, WCH)], pkw)
    for b in range(LA):
        fill(b, b)

    pltpu.sync_copy(zeros_hbm.at[pl.ds(zbase, _ZROWS_PER_TILE)],
                    acc_sh.at[pl.ds(zbase, _ZROWS_PER_TILE)])

    plsc.subcore_barrier()

    def slot(k, first_window):
        b = k % GB
        pltpu.make_async_copy(g_hbm.at[rowr[b]], bufs[b], gsem[b]).wait()
        pltpu.async_copy(bufs[b], acc_sh.at[colr[b]], ssem[b], add=True)
        if k + LA < WCH:
            b2 = (k + LA) % GB
            if first_window and k + LA < GB:
                fill(b2, k + LA)
            else:
                refill(b2, k + LA)

    for k in range(WCH):
        slot(k, True)

    def window(win, carry):
        pltpu.sync_copy(pk_hbm.at[w, pl.ds(win * WCH, WCH)], pkw)
        for j in range(LA):
            refill(j % GB, j)
        for k in range(WCH):
            slot(k, False)
        return carry

    lax.fori_loop(1, NWIN, window, 0)

    for b in range(GB):
        pltpu.make_async_copy(bufs[b], acc_sh.at[colr[b]], ssem[b]).wait()

    plsc.subcore_barrier()

    pltpu.sync_copy(acc_sh.at[pl.ds(zbase, _ZROWS_PER_TILE)],
                    out_hbm.at[c, pl.ds(zbase, _ZROWS_PER_TILE)])


_SPMM_SCRATCH = (
    [pltpu.VMEM((WCH, CHUNK), jnp.int32),
     pltpu.VMEM_SHARED((ACC_ROWS, D), jnp.float32)]
    + [pltpu.VMEM((CHUNK, D), jnp.float32) for _ in range(GB)]
    + [pltpu.VMEM((CHUNK,), jnp.int32) for _ in range(2 * GB)]
    + [pltpu.SemaphoreType.DMA for _ in range(2 * GB)]
)


@functools.partial(
    pl.kernel,
    out_type=jax.ShapeDtypeStruct((NC, ACC_ROWS, D), jnp.float32),
    mesh=_mesh,
    scratch_types=_SPMM_SCRATCH,
)
def _spmm_sc(g_hbm, pk_hbm, zeros_hbm, out_hbm, pkw, acc_sh, *rest):
    bufs = rest[0:GB]
    rowr = rest[GB:2 * GB]
    colr = rest[2 * GB:3 * GB]
    gsem = rest[3 * GB:4 * GB]
    ssem = rest[4 * GB:]
    c = lax.axis_index("c")
    s = lax.axis_index("s")
    w = c * NS + s
    _spmm_pipeline(g_hbm, out_hbm, pk_hbm, zeros_hbm, pkw, acc_sh,
                   bufs, rowr, colr, gsem, ssem, w, c, s)


@functools.partial(
    pl.kernel,
    out_type=[jax.ShapeDtypeStruct((NC, ACC_ROWS, D), jnp.float32),
              jax.ShapeDtypeStruct((NC, ACC_ROWS, D), jnp.float32)],
    mesh=_mesh,
    scratch_types=_SPMM_SCRATCH,
)
def _spmm2_sc(g1_hbm, g2_hbm, pk_hbm, zeros_hbm, out1_hbm, out2_hbm,
              pkw, acc_sh, *rest):
    bufs = rest[0:GB]
    rowr = rest[GB:2 * GB]
    colr = rest[2 * GB:3 * GB]
    gsem = rest[3 * GB:4 * GB]
    ssem = rest[4 * GB:]
    c = lax.axis_index("c")
    s = lax.axis_index("s")
    w = c * NS + s
    _spmm_pipeline(g1_hbm, out1_hbm, pk_hbm, zeros_hbm, pkw, acc_sh,
                   bufs, rowr, colr, gsem, ssem, w, c, s)
    _spmm_pipeline(g2_hbm, out2_hbm, pk_hbm, zeros_hbm, pkw, acc_sh,
                   bufs, rowr, colr, gsem, ssem, w, c, s)


DCHUNK = 128
DCH = PW // DCHUNK
DNB = 2


@functools.partial(
    pl.kernel,
    out_type=jax.ShapeDtypeStruct((NC, ACC_ROWS), jnp.float32),
    mesh=_mesh,
    scratch_types=(
        [pltpu.VMEM((DCH, DCHUNK), jnp.int32),
         pltpu.VMEM((DCHUNK,), jnp.float32),
         pltpu.VMEM((DCHUNK,), jnp.int32),
         pltpu.VMEM_SHARED((ACC_ROWS,), jnp.float32)]
        + [pltpu.VMEM((DCHUNK,), jnp.int32) for _ in range(DNB)]
        + [pltpu.SemaphoreType.DMA for _ in range(DNB)]
    ),
)
def _deg_sc(pk_hbm, ones_hbm, zeros1_hbm, out_hbm, pk_v, ones_v, row_x,
            acc_sh, *rest):
    colr = rest[:DNB]
    ssem = rest[DNB:]
    c = lax.axis_index("c")
    s = lax.axis_index("s")
    w = c * NS + s

    zbase = s * _ZROWS_PER_TILE
    pltpu.sync_copy(zeros1_hbm.at[pl.ds(zbase, _ZROWS_PER_TILE)],
                    acc_sh.at[pl.ds(zbase, _ZROWS_PER_TILE)])
    pltpu.sync_copy(pk_hbm.at[w], pk_v)
    pltpu.sync_copy(ones_hbm, ones_v)

    plsc.subcore_barrier()

    def outer(o, carry):
        for k in range(DNB):
            t = o * DNB + k

            @pl.when(t >= DNB)
            def _():
                pltpu.make_async_copy(ones_v, acc_sh.at[colr[k]],
                                      ssem[k]).wait()

            _unpack_chunk(pk_v, t, row_x, colr[k], nq=DCHUNK // 16)
            pltpu.async_copy(ones_v, acc_sh.at[colr[k]], ssem[k], add=True)
        return carry

    lax.fori_loop(0, DCH // DNB, outer, 0)

    for b in range(DNB):
        pltpu.make_async_copy(ones_v, acc_sh.at[colr[b]], ssem[b]).wait()

    plsc.subcore_barrier()

    pltpu.sync_copy(acc_sh.at[pl.ds(zbase, _ZROWS_PER_TILE)],
                    out_hbm.at[c, pl.ds(zbase, _ZROWS_PER_TILE)])


R = 2000
GRID = N // R


def _rowspec(d):
    return pl.BlockSpec((R, d), lambda i: (i, 0))


def _pairspec(d):
    return pl.BlockSpec((NC, R, d), lambda i: (0, i, 0))


def _fullspec(shape):
    nd = len(shape)
    return pl.BlockSpec(shape, lambda i, _n=nd: (0,) * _n)


def _colspec():
    return pl.BlockSpec((R, 1), lambda i: (i, 0))


def _matmul_t(a, w):
    return lax.dot_general(a, w, (((1,), (1,)), ((), ())),
                           preferred_element_type=jnp.float32)


def _tc_deg_body(degp, dis_o):
    deg = degp[0] + degp[1] + 1.0
    dis_o[...] = lax.rsqrt(deg)[:, None]


def _tc_deg(degp):
    return pl.pallas_call(
        _tc_deg_body,
        grid=(1,),
        in_specs=[_fullspec((NC, ACC_ROWS))],
        out_specs=_fullspec((ACC_ROWS, 1)),
        out_shape=jax.ShapeDtypeStruct((ACC_ROWS, 1), jnp.float32),
    )(degp)


def _tc_a_body(dis, x, w1, b1, xs_o, t0_o):
    xs_o[...] = x[...] * dis[...]
    t0_o[...] = _matmul_t(x[...], w1[...]) + b1[...][None, :]


def _tc_a(dis, x, w1, b1):
    return pl.pallas_call(
        _tc_a_body,
        grid=(GRID,),
        in_specs=[_colspec(), _rowspec(D),
                  _fullspec((D, D)), _fullspec((D,))],
        out_specs=[_rowspec(D), _rowspec(D)],
        out_shape=[jax.ShapeDtypeStruct((N, D), jnp.float32),
                   jax.ShapeDtypeStruct((N, D), jnp.float32)],
    )(dis, x, w1, b1)


def _tc_b_body(p1, dis, xs, w1, b1, t1_o, ys_o):
    y1 = (p1[0] + p1[1] + xs[...]) * dis[...]
    t1_o[...] = _matmul_t(y1, w1[...]) + b1[...][None, :]
    ys_o[...] = y1 * dis[...]


def _tc_b(p1, dis, xs, w1, b1):
    return pl.pallas_call(
        _tc_b_body,
        grid=(GRID,),
        in_specs=[_pairspec(D), _colspec(), _rowspec(D),
                  _fullspec((D, D)), _fullspec((D,))],
        out_specs=[_rowspec(D), _rowspec(D)],
        out_shape=[jax.ShapeDtypeStruct((N, D), jnp.float32),
                   jax.ShapeDtypeStruct((N, D), jnp.float32)],
    )(p1, dis, xs, w1, b1)


def _tc_c_body(p2, dis, ys, t0, t1, w12, b12, w20, b20, w21, b21, w22, b22,
               u0_o, z1s_o, z2s_o):
    disv = dis[...]
    y2 = (p2[0] + p2[1] + ys[...]) * disv
    t2 = _matmul_t(y2, w12[...]) + b12[...][None, :]
    h = jnp.maximum(jnp.concatenate([t0[...], t1[...], t2], axis=1), 0.0)
    u0_o[...] = _matmul_t(h, w20[...]) + b20[...][None, :]
    z1s_o[...] = (_matmul_t(h, w21[...]) + b21[...][None, :]) * disv
    z2s_o[...] = (_matmul_t(h, w22[...]) + b22[...][None, :]) * disv


def _tc_c(p2, dis, ys, t0, t1, w12, b12, w20, b20, w21, b21, w22, b22):
    return pl.pallas_call(
        _tc_c_body,
        grid=(GRID,),
        in_specs=[_pairspec(D), _colspec(), _rowspec(D), _rowspec(D),
                  _rowspec(D), _fullspec((D, D)), _fullspec((D,)),
                  _fullspec((D, 3 * D)), _fullspec((D,)),
                  _fullspec((D, 3 * D)), _fullspec((D,)),
                  _fullspec((D, 3 * D)), _fullspec((D,))],
        out_specs=[_rowspec(D), _rowspec(D), _rowspec(D)],
        out_shape=[jax.ShapeDtypeStruct((N, D), jnp.float32),
                   jax.ShapeDtypeStruct((N, D), jnp.float32),
                   jax.ShapeDtypeStruct((N, D), jnp.float32)],
    )(p2, dis, ys, t0, t1, w12, b12, w20, b20, w21, b21, w22, b22)


def _tc_d_body(p3, p4, dis, z1s, z2s, u1_o, vs_o):
    disv = dis[...]
    u1_o[...] = (p3[0] + p3[1] + z1s[...]) * disv
    vs_o[...] = (p4[0] + p4[1] + z2s[...]) * disv * disv


def _tc_d(p3, p4, dis, z1s, z2s):
    return pl.pallas_call(
        _tc_d_body,
        grid=(GRID,),
        in_specs=[_pairspec(D), _pairspec(D), _colspec(), _rowspec(D),
                  _rowspec(D)],
        out_specs=[_rowspec(D), _rowspec(D)],
        out_shape=[jax.ShapeDtypeStruct((N, D), jnp.float32),
                   jax.ShapeDtypeStruct((N, D), jnp.float32)],
    )(p3, p4, dis, z1s, z2s)


def _tc_e_body(p5, dis, vs, u0, u1, wf, bf, out_o):
    u2 = (p5[0] + p5[1] + vs[...]) * dis[...]
    wfm = wf[...]
    acc = _matmul_t(u0[...], wfm[:, 0:D])
    acc = acc + _matmul_t(u1[...], wfm[:, D:2 * D])
    acc = acc + _matmul_t(u2, wfm[:, 2 * D:3 * D])
    out_o[...] = acc + bf[...][None, :]


def _tc_e(p5, dis, vs, u0, u1, wf, bf):
    return pl.pallas_call(
        _tc_e_body,
        grid=(GRID,),
        in_specs=[_pairspec(D), _colspec(), _rowspec(D), _rowspec(D),
                  _rowspec(D), _fullspec((D, 3 * D)), _fullspec((D,))],
        out_specs=_rowspec(D),
        out_shape=jax.ShapeDtypeStruct((N, D), jnp.float32),
    )(p5, dis, vs, u0, u1, wf, bf)


def kernel(x, edge_index, W1_0, b1_0, W1_1, b1_1, W1_2, b1_2,
           W2_0, b2_0, W2_1, b2_1, W2_2, b2_2, Wf, bf):
    row = edge_index[0].astype(jnp.int32)
    col = edge_index[1].astype(jnp.int32)
    npad = EP - E
    packed = (row << SHIFT) | col
    pad_i = jnp.arange(npad, dtype=jnp.int32)
    pad_pk = (((pad_i * 197) % N) << SHIFT) | (N + pad_i % (ACC_ROWS - N))
    packed = jnp.concatenate([packed, pad_pk])
    packed = packed.reshape(CH * CHUNK, NW).T.reshape(NW, CH, CHUNK)

    zeros2 = jnp.zeros((ACC_ROWS, D), jnp.float32)
    zeros1 = jnp.zeros((ACC_ROWS,), jnp.float32)
    ones1 = jnp.ones((DCHUNK,), jnp.float32)

    degp = _deg_sc(packed.reshape(NW, DCH, DCHUNK), ones1, zeros1)
    dis = _tc_deg(degp)
    xs, t0 = _tc_a(dis, x, W1_0, b1_0)
    p1 = _spmm_sc(xs, packed, zeros2)
    t1, ys = _tc_b(p1, dis, xs, W1_1, b1_1)
    p2 = _spmm_sc(ys, packed, zeros2)
    u0, z1s, z2s = _tc_c(p2, dis, ys, t0, t1, W1_2, b1_2,
                         W2_0, b2_0, W2_1, b2_1, W2_2, b2_2)
    p3, p4 = _spmm2_sc(z1s, z2s, packed, zeros2)
    u1, vs = _tc_d(p3, p4, dis, z1s, z2s)
    p5 = _spmm_sc(vs, packed, zeros2)
    out = _tc_e(p5, dis, vs, u0, u1, Wf, bf)
    return out
